# baseline (speedup 1.0000x reference)
import jax
import jax.numpy as jnp
from jax.experimental import pallas as pl
from jax.experimental.pallas import tpu as pltpu

_BM = 1024
_NCHUNKS = 8


def _moe_dense_kernel(task_ref, x_ref, w_hbm, b_ref, o_ref, w_vmem, sems):
    n = o_ref.shape[1]
    cn = n // _NCHUNKS
    t = jnp.minimum(jnp.maximum(task_ref[0], 0), w_hbm.shape[0] - 1)

    @pl.when(pl.program_id(0) == 0)
    def _first_step():
        for kb in range(_NCHUNKS):
            pltpu.make_async_copy(
                w_hbm.at[t, :, pl.ds(kb * cn, cn)],
                w_vmem.at[:, pl.ds(kb * cn, cn)],
                sems.at[kb],
            ).start()
        for kb in range(_NCHUNKS):
            pltpu.make_async_copy(
                w_hbm.at[t, :, pl.ds(kb * cn, cn)],
                w_vmem.at[:, pl.ds(kb * cn, cn)],
                sems.at[kb],
            ).wait()
            o_ref[:, pl.ds(kb * cn, cn)] = (
                jnp.dot(
                    x_ref[...],
                    w_vmem[:, pl.ds(kb * cn, cn)],
                    preferred_element_type=jnp.float32,
                )
                + b_ref[0, 0, pl.ds(kb * cn, cn)]
            )

    @pl.when(pl.program_id(0) != 0)
    def _rest():
        o_ref[...] = (
            jnp.dot(x_ref[...], w_vmem[...], preferred_element_type=jnp.float32)
            + b_ref[0, 0]
        )


def kernel(inputs, kernel, bias, task_idx):
    m, k = inputs.shape
    n_tasks, _, n = kernel.shape
    t = jnp.asarray(task_idx, jnp.int32).reshape((1,))
    bias3 = bias.reshape(n_tasks, 1, n)

    def _expert(s):
        return jnp.minimum(jnp.maximum(s[0], 0), n_tasks - 1)

    out = pl.pallas_call(
        _moe_dense_kernel,
        grid_spec=pltpu.PrefetchScalarGridSpec(
            num_scalar_prefetch=1,
            grid=(m // _BM,),
            in_specs=[
                pl.BlockSpec((_BM, k), lambda i, s: (i, 0)),
                pl.BlockSpec(memory_space=pltpu.MemorySpace.HBM),
                pl.BlockSpec((1, 1, n), lambda i, s: (_expert(s), 0, 0)),
            ],
            out_specs=pl.BlockSpec((_BM, n), lambda i, s: (i, 0)),
            scratch_shapes=[
                pltpu.VMEM((k, n), jnp.float32),
                pltpu.SemaphoreType.DMA((_NCHUNKS,)),
            ],
        ),
        out_shape=jax.ShapeDtypeStruct((m, n), jnp.float32),
        compiler_params=pltpu.CompilerParams(
            dimension_semantics=("parallel",),
            vmem_limit_bytes=63 * 1024 * 1024,
        ),
    )(t, inputs, kernel, bias3)
    return out


# speedup vs baseline: 1.0537x; 1.0537x over previous
import jax
import jax.numpy as jnp
from jax.experimental import pallas as pl
from jax.experimental.pallas import tpu as pltpu

_BM = 1024


def _moe_dense_kernel(task_ref, x_ref, w_ref, b_ref, o_ref):
    del task_ref
    o_ref[...] = (
        jnp.dot(x_ref[...], w_ref[0], preferred_element_type=jnp.float32)
        + b_ref[0, 0]
    )


def kernel(inputs, kernel, bias, task_idx):
    m, k = inputs.shape
    n_tasks, _, n = kernel.shape
    t = jnp.asarray(task_idx, jnp.int32).reshape((1,))
    bias3 = bias.reshape(n_tasks, 1, n)

    def _expert(s):
        return jnp.minimum(jnp.maximum(s[0], 0), n_tasks - 1)
    out = pl.pallas_call(
        _moe_dense_kernel,
        grid_spec=pltpu.PrefetchScalarGridSpec(
            num_scalar_prefetch=1,
            grid=(m // _BM,),
            in_specs=[
                pl.BlockSpec((_BM, k), lambda i, s: (i, 0)),
                pl.BlockSpec((1, k, n), lambda i, s: (_expert(s), 0, 0)),
                pl.BlockSpec((1, 1, n), lambda i, s: (_expert(s), 0, 0)),
            ],
            out_specs=pl.BlockSpec((_BM, n), lambda i, s: (i, 0)),
        ),
        out_shape=jax.ShapeDtypeStruct((m, n), jnp.float32),
        compiler_params=pltpu.CompilerParams(
            dimension_semantics=("parallel",),
            vmem_limit_bytes=63 * 1024 * 1024,
        ),
    )(t, inputs, kernel, bias3)
    return out
